# Initial kernel scaffold; baseline (speedup 1.0000x reference)
import jax
import jax.numpy as jnp
from jax.experimental import pallas as pl


def kernel(x, edge_index, W0, a0, W1, a1):
    raise NotImplementedError("write your pallas kernel here")


# speedup vs baseline: 6.7858x; 6.7858x over previous
import functools

import jax
import jax.numpy as jnp
from jax import lax
from jax.experimental import pallas as pl
from jax.experimental.pallas import tpu as pltpu
from jax.experimental.pallas import tpu_sc as plsc

N = 10000
D = 256
H = 128
NP = 10240
RPT = NP // 16
E = 160000
EC = 128
ER = 1280
EP = ER * EC
ERT = ER // 16
RB = 1024
NB = NP // RB

_mesh = plsc.VectorSubcoreMesh(core_axis_name="c", subcore_axis_name="s")


def _deg_body(dst_hbm, cnt_hbm, dst_v, ones_v, zeros_v, acc):
    c = lax.axis_index("c")
    s = lax.axis_index("s")
    for i in range(RPT // 16):
        zeros_v[pl.ds(i * 16, 16)] = jnp.zeros((16,), jnp.float32)
    for i in range(EC // 16):
        ones_v[pl.ds(i * 16, 16)] = jnp.ones((16,), jnp.float32)
    pltpu.sync_copy(zeros_v, acc.at[pl.ds(s * RPT, RPT)])
    pltpu.sync_copy(dst_hbm.at[pl.ds(s * ERT, ERT)], dst_v)
    plsc.subcore_barrier()

    def step(j, carry):
        pltpu.sync_copy(ones_v, acc.at[dst_v.at[j]], add=True)
        return carry

    lax.fori_loop(0, ERT, step, 0)
    plsc.subcore_barrier()

    @pl.when(c == 0)
    def _():
        pltpu.sync_copy(acc.at[pl.ds(s * RPT, RPT)],
                        cnt_hbm.at[pl.ds(s * RPT, RPT)])


def _sc_degree(dst2d):
    return pl.kernel(
        _deg_body,
        out_type=jax.ShapeDtypeStruct((NP,), jnp.float32),
        mesh=_mesh,
        scratch_types=[
            pltpu.VMEM((ERT, EC), jnp.int32),
            pltpu.VMEM((EC,), jnp.float32),
            pltpu.VMEM((RPT,), jnp.float32),
            pltpu.VMEM_SHARED((NP,), jnp.float32),
        ],
    )(dst2d)


def _prop_body(glo, ghi, src_hbm, dst_hbm, ulo, uhi,
               src_v, dst_v, rows_v, acc, sem):
    c = lax.axis_index("c")
    s = lax.axis_index("s")

    def run(g_hbm, u_hbm):
        pltpu.sync_copy(g_hbm.at[pl.ds(s * RPT, RPT)],
                        acc.at[pl.ds(s * RPT, RPT)])
        pltpu.sync_copy(src_hbm.at[pl.ds(s * ERT, ERT)], src_v)
        pltpu.sync_copy(dst_hbm.at[pl.ds(s * ERT, ERT)], dst_v)
        plsc.subcore_barrier()

        def step(j, carry):
            pltpu.async_copy(g_hbm.at[src_v.at[j]], rows_v, sem).wait()
            pltpu.sync_copy(rows_v, acc.at[dst_v.at[j]], add=True)
            return carry

        lax.fori_loop(0, ERT, step, 0)
        plsc.subcore_barrier()
        pltpu.sync_copy(acc.at[pl.ds(s * RPT, RPT)],
                        u_hbm.at[pl.ds(s * RPT, RPT)])

    @pl.when(c == 0)
    def _():
        run(glo, ulo)

    @pl.when(c == 1)
    def _():
        run(ghi, uhi)


def _sc_prop(glo, ghi, src2d, dst2d):
    return pl.kernel(
        _prop_body,
        out_type=(jax.ShapeDtypeStruct((NP, H), jnp.float32),
                  jax.ShapeDtypeStruct((NP, H), jnp.float32)),
        mesh=_mesh,
        scratch_types=[
            pltpu.VMEM((ERT, EC), jnp.int32),
            pltpu.VMEM((ERT, EC), jnp.int32),
            pltpu.VMEM((EC, H), jnp.float32),
            pltpu.VMEM_SHARED((NP, H), jnp.float32),
            pltpu.SemaphoreType.DMA,
        ],
    )(glo, ghi, src2d, dst2d)


def _tc_in_body(x_ref, deg_ref, w_ref, glo_ref, ghi_ref):
    dis = lax.rsqrt(deg_ref[...] + 1.0)
    h = jnp.dot(x_ref[...], w_ref[...],
                preferred_element_type=jnp.float32)
    g = h * dis
    glo_ref[...] = g[:, :H]
    ghi_ref[...] = g[:, H:]


def _tc_in(xp, deg2, W0):
    return pl.pallas_call(
        _tc_in_body,
        grid=(NB,),
        in_specs=[
            pl.BlockSpec((RB, D), lambda i: (i, 0)),
            pl.BlockSpec((RB, 1), lambda i: (i, 0)),
            pl.BlockSpec((D, D), lambda i: (0, 0)),
        ],
        out_specs=(pl.BlockSpec((RB, H), lambda i: (i, 0)),
                   pl.BlockSpec((RB, H), lambda i: (i, 0))),
        out_shape=(jax.ShapeDtypeStruct((NP, H), jnp.float32),
                   jax.ShapeDtypeStruct((NP, H), jnp.float32)),
    )(xp, deg2, W0)


def _tc_mid_body(ulo_ref, uhi_ref, deg_ref, w_ref, a_ref, glo_ref, ghi_ref):
    dis = lax.rsqrt(deg_ref[...] + 1.0)
    u = jnp.concatenate([ulo_ref[...], uhi_ref[...]], axis=1)
    p = u * dis
    z = jnp.where(p >= 0, p, a_ref[...] * p)
    g = jnp.dot(z, w_ref[...], preferred_element_type=jnp.float32) * dis
    glo_ref[...] = g[:, :H]
    ghi_ref[...] = g[:, H:]


def _tc_mid(ulo, uhi, deg2, W1, a0r):
    return pl.pallas_call(
        _tc_mid_body,
        grid=(NB,),
        in_specs=[
            pl.BlockSpec((RB, H), lambda i: (i, 0)),
            pl.BlockSpec((RB, H), lambda i: (i, 0)),
            pl.BlockSpec((RB, 1), lambda i: (i, 0)),
            pl.BlockSpec((D, D), lambda i: (0, 0)),
            pl.BlockSpec((1, D), lambda i: (0, 0)),
        ],
        out_specs=(pl.BlockSpec((RB, H), lambda i: (i, 0)),
                   pl.BlockSpec((RB, H), lambda i: (i, 0))),
        out_shape=(jax.ShapeDtypeStruct((NP, H), jnp.float32),
                   jax.ShapeDtypeStruct((NP, H), jnp.float32)),
    )(ulo, uhi, deg2, W1, a0r)


def _tc_out_body(ulo_ref, uhi_ref, deg_ref, a_ref, out_ref):
    dis = lax.rsqrt(deg_ref[...] + 1.0)
    u = jnp.concatenate([ulo_ref[...], uhi_ref[...]], axis=1)
    p = u * dis
    out_ref[...] = jnp.where(p >= 0, p, a_ref[...] * p)


def _tc_out(ulo, uhi, deg2, a1r):
    return pl.pallas_call(
        _tc_out_body,
        grid=(NB,),
        in_specs=[
            pl.BlockSpec((RB, H), lambda i: (i, 0)),
            pl.BlockSpec((RB, H), lambda i: (i, 0)),
            pl.BlockSpec((RB, 1), lambda i: (i, 0)),
            pl.BlockSpec((1, D), lambda i: (0, 0)),
        ],
        out_specs=pl.BlockSpec((RB, D), lambda i: (i, 0)),
        out_shape=jax.ShapeDtypeStruct((NP, D), jnp.float32),
    )(ulo, uhi, deg2, a1r)


@jax.jit
def kernel(x, edge_index, W0, a0, W1, a1):
    xp = jnp.pad(x, ((0, NP - N), (0, 0)))
    src = jnp.pad(edge_index[0], (0, EP - E), constant_values=N)
    dst = jnp.pad(edge_index[1], (0, EP - E), constant_values=N)
    src2d = src.reshape(ER, EC)
    dst2d = dst.reshape(ER, EC)

    cnt = _sc_degree(dst2d)
    deg2 = cnt.reshape(NP, 1)

    glo, ghi = _tc_in(xp, deg2, W0)
    ulo, uhi = _sc_prop(glo, ghi, src2d, dst2d)
    vlo, vhi = _tc_mid(ulo, uhi, deg2, W1, a0.reshape(1, D))
    wlo, whi = _sc_prop(vlo, vhi, src2d, dst2d)
    out = _tc_out(wlo, whi, deg2, a1.reshape(1, D))
    return out[:N]


# speedup vs baseline: 7.9551x; 1.1723x over previous
import jax
import jax.numpy as jnp
from jax import lax
from jax.experimental import pallas as pl
from jax.experimental.pallas import tpu as pltpu
from jax.experimental.pallas import tpu_sc as plsc

N = 10000
D = 256
H = 128
NP = 10240
RPT = NP // 16
E = 160000
EC = 128
ER = 1280
EP = ER * EC
ERT = ER // 16
RB = 1024
NB = NP // RB
NBUF = 2
ERH = 40

_mesh = plsc.VectorSubcoreMesh(core_axis_name="c", subcore_axis_name="s")


def _deg_body(dst_hbm, cnt_hbm, dst_v, ones_v, zeros_v, acc):
    c = lax.axis_index("c")
    s = lax.axis_index("s")
    for i in range(RPT // 16):
        zeros_v[pl.ds(i * 16, 16)] = jnp.zeros((16,), jnp.float32)
    for i in range(EC // 16):
        ones_v[pl.ds(i * 16, 16)] = jnp.ones((16,), jnp.float32)
    pltpu.sync_copy(zeros_v, acc.at[pl.ds(s * RPT, RPT)])
    pltpu.sync_copy(dst_hbm.at[pl.ds(s * ERT, ERT)], dst_v)
    plsc.subcore_barrier()

    def step(j, carry):
        pltpu.sync_copy(ones_v, acc.at[dst_v.at[j]], add=True)
        return carry

    lax.fori_loop(0, ERT, step, 0)
    plsc.subcore_barrier()

    @pl.when(c == 0)
    def _():
        pltpu.sync_copy(acc.at[pl.ds(s * RPT, RPT)],
                        cnt_hbm.at[pl.ds(s * RPT, RPT)])


def _sc_degree(dst2d):
    return pl.kernel(
        _deg_body,
        out_type=jax.ShapeDtypeStruct((NP,), jnp.float32),
        mesh=_mesh,
        scratch_types=[
            pltpu.VMEM((ERT, EC), jnp.int32),
            pltpu.VMEM((EC,), jnp.float32),
            pltpu.VMEM((RPT,), jnp.float32),
            pltpu.VMEM_SHARED((NP,), jnp.float32),
        ],
    )(dst2d)


def _prop_body(glo, ghi, src_hbm, dst_hbm, ulo, uhi,
               src_v, dst_v, rows0, rows1, acc, sem):
    c = lax.axis_index("c")
    s = lax.axis_index("s")
    rows = (rows0, rows1)

    def run(g_hbm, u_hbm):
        pltpu.sync_copy(g_hbm.at[pl.ds(s * RPT, RPT)],
                        acc.at[pl.ds(s * RPT, RPT)])
        plsc.subcore_barrier()

        for half in range(ERT // ERH):
            base = s * ERT + half * ERH
            pltpu.sync_copy(src_hbm.at[pl.ds(base, ERH)], src_v)
            pltpu.sync_copy(dst_hbm.at[pl.ds(base, ERH)], dst_v)

            def gather(j, b):
                return pltpu.make_async_copy(g_hbm.at[src_v.at[j]],
                                             rows[b], sem.at[b])

            for b in range(NBUF):
                gather(b, b).start()

            def step(g, carry):
                for b in range(NBUF):
                    j = g * NBUF + b
                    gather(j, b).wait()
                    pltpu.sync_copy(rows[b], acc.at[dst_v.at[j]], add=True)

                    @pl.when(j + NBUF < ERH)
                    def _():
                        gather(j + NBUF, b).start()

                return carry

            lax.fori_loop(0, ERH // NBUF, step, 0)
        plsc.subcore_barrier()
        pltpu.sync_copy(acc.at[pl.ds(s * RPT, RPT)],
                        u_hbm.at[pl.ds(s * RPT, RPT)])

    @pl.when(c == 0)
    def _():
        run(glo, ulo)

    @pl.when(c == 1)
    def _():
        run(ghi, uhi)


def _sc_prop(glo, ghi, src2d, dst2d):
    return pl.kernel(
        _prop_body,
        out_type=(jax.ShapeDtypeStruct((NP, H), jnp.float32),
                  jax.ShapeDtypeStruct((NP, H), jnp.float32)),
        mesh=_mesh,
        scratch_types=[
            pltpu.VMEM((ERH, EC), jnp.int32),
            pltpu.VMEM((ERH, EC), jnp.int32),
            pltpu.VMEM((EC, H), jnp.float32),
            pltpu.VMEM((EC, H), jnp.float32),
            pltpu.VMEM_SHARED((NP, H), jnp.float32),
            pltpu.SemaphoreType.DMA((NBUF,)),
        ],
    )(glo, ghi, src2d, dst2d)


def _tc_in_body(x_ref, deg_ref, w_ref, glo_ref, ghi_ref):
    dis = lax.rsqrt(deg_ref[...] + 1.0)
    h = jnp.dot(x_ref[...], w_ref[...],
                preferred_element_type=jnp.float32)
    g = h * dis
    glo_ref[...] = g[:, :H]
    ghi_ref[...] = g[:, H:]


def _tc_in(xp, deg2, W0):
    return pl.pallas_call(
        _tc_in_body,
        grid=(NB,),
        in_specs=[
            pl.BlockSpec((RB, D), lambda i: (i, 0)),
            pl.BlockSpec((RB, 1), lambda i: (i, 0)),
            pl.BlockSpec((D, D), lambda i: (0, 0)),
        ],
        out_specs=(pl.BlockSpec((RB, H), lambda i: (i, 0)),
                   pl.BlockSpec((RB, H), lambda i: (i, 0))),
        out_shape=(jax.ShapeDtypeStruct((NP, H), jnp.float32),
                   jax.ShapeDtypeStruct((NP, H), jnp.float32)),
    )(xp, deg2, W0)


def _tc_mid_body(ulo_ref, uhi_ref, deg_ref, w_ref, a_ref, glo_ref, ghi_ref):
    dis = lax.rsqrt(deg_ref[...] + 1.0)
    u = jnp.concatenate([ulo_ref[...], uhi_ref[...]], axis=1)
    p = u * dis
    z = jnp.where(p >= 0, p, a_ref[...] * p)
    g = jnp.dot(z, w_ref[...], preferred_element_type=jnp.float32) * dis
    glo_ref[...] = g[:, :H]
    ghi_ref[...] = g[:, H:]


def _tc_mid(ulo, uhi, deg2, W1, a0r):
    return pl.pallas_call(
        _tc_mid_body,
        grid=(NB,),
        in_specs=[
            pl.BlockSpec((RB, H), lambda i: (i, 0)),
            pl.BlockSpec((RB, H), lambda i: (i, 0)),
            pl.BlockSpec((RB, 1), lambda i: (i, 0)),
            pl.BlockSpec((D, D), lambda i: (0, 0)),
            pl.BlockSpec((1, D), lambda i: (0, 0)),
        ],
        out_specs=(pl.BlockSpec((RB, H), lambda i: (i, 0)),
                   pl.BlockSpec((RB, H), lambda i: (i, 0))),
        out_shape=(jax.ShapeDtypeStruct((NP, H), jnp.float32),
                   jax.ShapeDtypeStruct((NP, H), jnp.float32)),
    )(ulo, uhi, deg2, W1, a0r)


def _tc_out_body(ulo_ref, uhi_ref, deg_ref, a_ref, out_ref):
    dis = lax.rsqrt(deg_ref[...] + 1.0)
    u = jnp.concatenate([ulo_ref[...], uhi_ref[...]], axis=1)
    p = u * dis
    out_ref[...] = jnp.where(p >= 0, p, a_ref[...] * p)


def _tc_out(ulo, uhi, deg2, a1r):
    return pl.pallas_call(
        _tc_out_body,
        grid=(NB,),
        in_specs=[
            pl.BlockSpec((RB, H), lambda i: (i, 0)),
            pl.BlockSpec((RB, H), lambda i: (i, 0)),
            pl.BlockSpec((RB, 1), lambda i: (i, 0)),
            pl.BlockSpec((1, D), lambda i: (0, 0)),
        ],
        out_specs=pl.BlockSpec((RB, D), lambda i: (i, 0)),
        out_shape=jax.ShapeDtypeStruct((NP, D), jnp.float32),
    )(ulo, uhi, deg2, a1r)


@jax.jit
def kernel(x, edge_index, W0, a0, W1, a1):
    xp = jnp.pad(x, ((0, NP - N), (0, 0)))
    src = jnp.pad(edge_index[0], (0, EP - E), constant_values=N)
    dst = jnp.pad(edge_index[1], (0, EP - E), constant_values=N)
    src2d = src.reshape(ER, EC)
    dst2d = dst.reshape(ER, EC)

    cnt = _sc_degree(dst2d)
    deg2 = cnt.reshape(NP, 1)

    glo, ghi = _tc_in(xp, deg2, W0)
    ulo, uhi = _sc_prop(glo, ghi, src2d, dst2d)
    vlo, vhi = _tc_mid(ulo, uhi, deg2, W1, a0.reshape(1, D))
    wlo, whi = _sc_prop(vlo, vhi, src2d, dst2d)
    return _tc_out(wlo, whi, deg2, a1.reshape(1, D))[:N]
